# baseline (speedup 1.0000x reference)
import functools

import jax
import jax.numpy as jnp
from jax import lax
from jax.experimental import pallas as pl
from jax.experimental.pallas import tpu as pltpu
from jax.experimental.pallas import tpu_sc as plsc

B = 16384
HALF = B // 2
K = 26
VOCAB = 100
VPAD = 128
DIM = 128

NC = 2
NS = 16
NW = NC * NS
RPW = HALF // NW
LANES = 16

NQ = 4
RPQ = RPW // NQ
GPQ = RPQ // LANES

_mesh = plsc.VectorSubcoreMesh(
    core_axis_name="c", subcore_axis_name="s", num_cores=NC, num_subcores=NS
)


def _make_hist(col0):
    @functools.partial(
        pl.kernel,
        out_type=jax.ShapeDtypeStruct((HALF, VPAD), jnp.float32),
        mesh=_mesh,
        scratch_types=[
            pltpu.VMEM((K, RPW), jnp.int32),
            pltpu.VMEM((K, RPW), jnp.float32),
            [pltpu.VMEM((RPQ, VPAD), jnp.float32) for _ in range(NQ)],
            pltpu.SemaphoreType.DMA,
            pltpu.SemaphoreType.DMA,
        ],
        compiler_params=pltpu.CompilerParams(
            use_tc_tiling_on_sc=False, needs_layout_passes=False
        ),
    )
    def _hist_kernel(idx_hbm, vals_hbm, h_hbm, idx_v, vals_v, h4, sem1, sem2):
        wid = lax.axis_index("s") * NC + lax.axis_index("c")
        row0 = wid * RPW
        src0 = col0 + row0
        cp1 = pltpu.async_copy(idx_hbm.at[:, pl.ds(src0, RPW)], idx_v, sem1)
        cp2 = pltpu.async_copy(vals_hbm.at[:, pl.ds(src0, RPW)], vals_v, sem2)

        zeros16 = jnp.zeros((LANES,), jnp.float32)

        @plsc.parallel_loop(0, RPQ, step=1, unroll=2)
        def _zero_body(r):
            for q in range(NQ):
                for c in range(0, VPAD, LANES):
                    h4[q][r, pl.ds(c, LANES)] = zeros16

        cp1.wait()
        cp2.wait()

        lane = lax.iota(jnp.int32, LANES)

        @plsc.parallel_loop(0, GPQ, step=1, unroll=2)
        def _scatter_body(g):
            rows = g * LANES + lane
            for k in range(K):
                for q in range(NQ):
                    col = q * RPQ + g * LANES
                    iv = idx_v[k, pl.ds(col, LANES)]
                    vv = vals_v[k, pl.ds(col, LANES)]
                    plsc.addupdate_scatter(h4[q], [rows, iv], vv)

        for q in range(NQ):
            pltpu.sync_copy(h4[q], h_hbm.at[pl.ds(row0 + q * RPQ, RPQ), :])

    return _hist_kernel


_hist0 = _make_hist(0)
_hist1 = _make_hist(HALF)


def _mm_body_first(h_ref, w_ref, o_ref):
    w = jnp.concatenate(
        [w_ref[:], jnp.zeros((VPAD - VOCAB, DIM), jnp.float32)], axis=0
    )
    o_ref[:] = jnp.dot(h_ref[:], w, preferred_element_type=jnp.float32)


_mm_first = pl.pallas_call(
    _mm_body_first,
    grid=(2,),
    in_specs=[
        pl.BlockSpec((HALF // 2, VPAD), lambda i: (i, 0)),
        pl.BlockSpec((VOCAB, DIM), lambda i: (0, 0)),
    ],
    out_specs=pl.BlockSpec((HALF // 2, DIM), lambda i: (i, 0)),
    out_shape=jax.ShapeDtypeStruct((B, DIM), jnp.float32),
)


def _mm_body_second(h_ref, w_ref, prev_ref, o_ref):
    del prev_ref
    w = jnp.concatenate(
        [w_ref[:], jnp.zeros((VPAD - VOCAB, DIM), jnp.float32)], axis=0
    )
    o_ref[:] = jnp.dot(h_ref[:], w, preferred_element_type=jnp.float32)


_mm_second = pl.pallas_call(
    _mm_body_second,
    grid=(2,),
    in_specs=[
        pl.BlockSpec((HALF // 2, VPAD), lambda i: (i, 0)),
        pl.BlockSpec((VOCAB, DIM), lambda i: (0, 0)),
        pl.BlockSpec(memory_space=pl.ANY),
    ],
    out_specs=pl.BlockSpec((HALF // 2, DIM), lambda i: (i + 2, 0)),
    out_shape=jax.ShapeDtypeStruct((B, DIM), jnp.float32),
    input_output_aliases={2: 0},
)


def kernel(idx, vals, weight):
    idx_t = idx.astype(jnp.int32).T
    vals_t = vals.T
    h0 = _hist0(idx_t, vals_t)
    h1 = _hist1(idx_t, vals_t)
    out = _mm_first(h0, weight)
    out = _mm_second(h1, weight, out)
    return out


# speedup vs baseline: 1.1287x; 1.1287x over previous
import functools

import jax
import jax.numpy as jnp
from jax import lax
from jax.experimental import pallas as pl
from jax.experimental.pallas import tpu as pltpu
from jax.experimental.pallas import tpu_sc as plsc

B = 16384
K = 26
VOCAB = 100
VPAD = 128
DIM = 128

NC = 2
NS = 16
NW = NC * NS
RPW = B // NW
LANES = 16

NQ = 4
RPQ = RPW // NQ
GPQ = RPQ // LANES

_mesh = plsc.VectorSubcoreMesh(
    core_axis_name="c", subcore_axis_name="s", num_cores=NC, num_subcores=NS
)


@functools.partial(
    pl.kernel,
    out_type=jax.ShapeDtypeStruct((B, VPAD), jnp.float32),
    mesh=_mesh,
    scratch_types=[
        pltpu.VMEM((K, RPW), jnp.int32),
        pltpu.VMEM((K, RPW), jnp.float32),
        [pltpu.VMEM((RPQ, VPAD), jnp.float32) for _ in range(NQ)],
        pltpu.SemaphoreType.DMA,
        pltpu.SemaphoreType.DMA,
        pltpu.SemaphoreType.DMA,
    ],
    compiler_params=pltpu.CompilerParams(
        use_tc_tiling_on_sc=False, needs_layout_passes=False
    ),
)
def _hist_kernel(idx_hbm, vals_hbm, h_hbm, idx_v, vals_v, h4, sem1, sem2, sem3):
    wid = lax.axis_index("s") * NC + lax.axis_index("c")
    row0 = wid * RPW
    cp1 = pltpu.async_copy(idx_hbm.at[:, pl.ds(row0, RPW)], idx_v, sem1)
    cp2 = pltpu.async_copy(vals_hbm.at[:, pl.ds(row0, RPW)], vals_v, sem2)

    zeros16 = jnp.zeros((LANES,), jnp.float32)

    @plsc.parallel_loop(0, RPQ, step=1, unroll=2)
    def _zero_body(r):
        for q in range(NQ):
            for c in range(0, VPAD, LANES):
                h4[q][r, pl.ds(c, LANES)] = zeros16

    cp1.wait()
    cp2.wait()

    lane = lax.iota(jnp.int32, LANES)

    @plsc.parallel_loop(0, GPQ, step=1, unroll=2)
    def _scatter_body(g):
        rows = g * LANES + lane
        for k in range(K):
            for q in range(NQ):
                col = q * RPQ + g * LANES
                iv = idx_v[k, pl.ds(col, LANES)]
                vv = vals_v[k, pl.ds(col, LANES)]
                plsc.addupdate_scatter(h4[q], [rows, iv], vv)

    cps = [
        pltpu.async_copy(
            h4[q], h_hbm.at[pl.ds(row0 + q * RPQ, RPQ), :], sem3
        )
        for q in range(NQ)
    ]
    for cp in cps:
        cp.wait()


_BM = 8192


def _mm_body(h_ref, w_ref, o_ref):
    w = jnp.concatenate(
        [w_ref[:], jnp.zeros((VPAD - VOCAB, DIM), jnp.float32)], axis=0
    )
    o_ref[:] = jnp.dot(h_ref[:], w, preferred_element_type=jnp.float32)


_matmul = pl.pallas_call(
    _mm_body,
    grid=(B // _BM,),
    in_specs=[
        pl.BlockSpec((_BM, VPAD), lambda i: (i, 0)),
        pl.BlockSpec((VOCAB, DIM), lambda i: (0, 0)),
    ],
    out_specs=pl.BlockSpec((_BM, DIM), lambda i: (i, 0)),
    out_shape=jax.ShapeDtypeStruct((B, DIM), jnp.float32),
)


def kernel(idx, vals, weight):
    idx_t = idx.astype(jnp.int32).T
    vals_t = vals.T
    h = _hist_kernel(idx_t, vals_t)
    return _matmul(h, weight)
